# baseline (speedup 1.0000x reference)
import jax
import jax.numpy as jnp
from jax import lax
from jax.experimental import pallas as pl
from jax.experimental.pallas import tpu as pltpu

E = 16
D = 768
H = 3072
N = 32
NSL1 = 8
NSL2 = 6
HS1 = H // NSL1
DS = D // NSL2
JPE = NSL1 + NSL2
T = E * JPE
W = JPE
B1 = 9
B2 = 7


def _w1_copy(w1_hbm, ring, sems, q, r):
    i = q * NSL1 + r
    return pltpu.make_async_copy(
        w1_hbm.at[pl.ds(i * HS1, HS1), :], ring.at[i % B1], sems.at[i % B1])


def _w2_copy(w2_hbm, ring, sems, q, r2):
    i = q * NSL2 + r2
    return pltpu.make_async_copy(
        w2_hbm.at[pl.ds(i * DS, DS), :], ring.at[i % B2], sems.at[i % B2])


def _moe_kernel(x_ref, wg_ref, bg_ref, b1_ref, b2_ref, w1_hbm, w2_hbm,
                out_ref, w1ring, w2ring, hs_ref, sem1, sem2):
    logits = lax.dot_general(
        x_ref[...], wg_ref[...], (((1,), (1,)), ((), ())),
        preferred_element_type=jnp.float32) + bg_ref[...]
    col = lax.broadcasted_iota(jnp.int32, (N, E), 1)
    m1 = jnp.max(logits, axis=-1, keepdims=True)
    i1 = jnp.min(jnp.where(logits == m1, col, E), axis=-1, keepdims=True)
    masked = jnp.where(col == i1, -jnp.inf, logits)
    m2 = jnp.max(masked, axis=-1, keepdims=True)
    i2 = jnp.min(jnp.where(masked == m2, col, E), axis=-1, keepdims=True)
    wa = 1.0 / (1.0 + jnp.exp(m2 - m1))
    wb = 1.0 - wa

    xb = x_ref[...].astype(jnp.bfloat16)

    for t in range(W):
        q, r = divmod(t, JPE)
        if r < NSL1:
            _w1_copy(w1_hbm, w1ring, sem1, q, r).start()
        else:
            _w2_copy(w2_hbm, w2ring, sem2, q, r - NSL1).start()

    def body(t, p):
        q = t // JPE
        r = t - q * JPE
        scol = (wa * (i1 == q).astype(jnp.float32)
                + wb * (i2 == q).astype(jnp.float32))

        for _ in range(2):
            ok = jnp.logical_and(p < T, p - t <= W)
            qn = p // JPE
            rn = p - qn * JPE

            @pl.when(jnp.logical_and(ok, rn < NSL1))
            def _issue1():
                _w1_copy(w1_hbm, w1ring, sem1, qn, rn).start()

            @pl.when(jnp.logical_and(ok, rn >= NSL1))
            def _issue2():
                _w2_copy(w2_hbm, w2ring, sem2, qn, rn - NSL1).start()

            p = jnp.where(ok, p + 1, p)

        @pl.when(r < NSL1)
        def _up_proj():
            i = q * NSL1 + r
            slot = i % B1
            _w1_copy(w1_hbm, w1ring, sem1, q, r).wait()
            h = lax.dot_general(
                xb, w1ring[slot].astype(jnp.bfloat16), (((1,), (1,)), ((), ())),
                preferred_element_type=jnp.float32)
            h = h + b1_ref[i]
            h = jnp.maximum(h, 0.0) * scol
            hs_ref[r] = h.astype(jnp.bfloat16)

        @pl.when(r >= NSL1)
        def _down_proj():
            r2 = r - NSL1
            i = q * NSL2 + r2
            slot = i % B2
            _w2_copy(w2_hbm, w2ring, sem2, q, r2).wait()
            w2s = w2ring[slot].astype(jnp.bfloat16)
            pb = scol * b2_ref[i]
            for k in range(NSL1):
                pb = pb + lax.dot_general(
                    hs_ref[k], w2s[:, k * HS1:(k + 1) * HS1],
                    (((1,), (1,)), ((), ())),
                    preferred_element_type=jnp.float32)
            prev = jnp.where(q == 0, jnp.zeros((N, DS), jnp.float32),
                             out_ref[r2])
            out_ref[r2] = prev + pb

        return p

    lax.fori_loop(0, T, body, jnp.int32(W))


@jax.jit
def _moe(x, Wg, bg2, W1, b1, W2, b2):
    return pl.pallas_call(
        _moe_kernel,
        in_specs=[
            pl.BlockSpec(memory_space=pltpu.VMEM),
            pl.BlockSpec(memory_space=pltpu.VMEM),
            pl.BlockSpec(memory_space=pltpu.VMEM),
            pl.BlockSpec(memory_space=pltpu.VMEM),
            pl.BlockSpec(memory_space=pltpu.VMEM),
            pl.BlockSpec(memory_space=pl.ANY),
            pl.BlockSpec(memory_space=pl.ANY),
        ],
        out_specs=pl.BlockSpec(memory_space=pltpu.VMEM),
        out_shape=jax.ShapeDtypeStruct((NSL2, N, DS), jnp.float32),
        scratch_shapes=[
            pltpu.VMEM((B1, HS1, D), jnp.float32),
            pltpu.VMEM((B2, DS, H), jnp.float32),
            pltpu.VMEM((NSL1, N, HS1), jnp.bfloat16),
            pltpu.SemaphoreType.DMA((B1,)),
            pltpu.SemaphoreType.DMA((B2,)),
        ],
    )(x, Wg, bg2, b1, b2, W1, W2)


def kernel(x, Wg, bg, W1, b1, W2, b2):
    out = _moe(x, Wg, bg.reshape(1, E),
               W1.reshape(E * H, D), b1.reshape(E * NSL1, 1, HS1),
               W2.reshape(E * D, H), b2.reshape(E * NSL2, 1, DS))
    return out.transpose(1, 0, 2).reshape(N, D)


# speedup vs baseline: 1.0081x; 1.0081x over previous
import jax
import jax.numpy as jnp
from jax import lax
from jax.experimental import pallas as pl
from jax.experimental.pallas import tpu as pltpu

E = 16
D = 768
H = 3072
N = 32
S1 = 8
S2 = 6
HS1 = H // S1
DS = D // S2


def _moe_kernel(*refs):
    (x_ref, wg_ref, bg_ref, b1_ref, b2_ref), w_refs, (out_ref,), \
        (hs_ref, w1s_ref, w2s_ref, i1s_ref, i2s_ref) = \
        refs[:5], refs[5:5 + S1 + S2], refs[5 + S1 + S2:6 + S1 + S2], \
        refs[6 + S1 + S2:]
    w1_refs = w_refs[:S1]
    w2_refs = w_refs[S1:]
    e = pl.program_id(0)
    first = e == 0

    @pl.when(first)
    def _gate():
        logits = lax.dot_general(
            x_ref[...], wg_ref[...], (((1,), (1,)), ((), ())),
            preferred_element_type=jnp.float32) + bg_ref[...]
        col = lax.broadcasted_iota(jnp.int32, (N, E), 1)
        m1 = jnp.max(logits, axis=-1, keepdims=True)
        i1 = jnp.min(jnp.where(logits == m1, col, E), axis=-1, keepdims=True)
        masked = jnp.where(col == i1, -jnp.inf, logits)
        m2 = jnp.max(masked, axis=-1, keepdims=True)
        i2 = jnp.min(jnp.where(masked == m2, col, E), axis=-1, keepdims=True)
        w1 = 1.0 / (1.0 + jnp.exp(m2 - m1))
        w1s_ref[...] = w1
        w2s_ref[...] = 1.0 - w1
        i1s_ref[...] = i1
        i2s_ref[...] = i2

    scol = (w1s_ref[...] * (i1s_ref[...] == e).astype(jnp.float32)
            + w2s_ref[...] * (i2s_ref[...] == e).astype(jnp.float32))

    xb = x_ref[...].astype(jnp.bfloat16)
    for k in range(S1):
        h = lax.dot_general(
            xb, w1_refs[k][0].astype(jnp.bfloat16), (((1,), (1,)), ((), ())),
            preferred_element_type=jnp.float32)
        h = h + b1_ref[0, :, k * HS1:(k + 1) * HS1]
        h = jnp.maximum(h, 0.0) * scol
        hs_ref[:, k * HS1:(k + 1) * HS1] = h.astype(jnp.bfloat16)

    hs = hs_ref[...]
    for m in range(S2):
        pb = lax.dot_general(
            hs, w2_refs[m][0].astype(jnp.bfloat16), (((1,), (1,)), ((), ())),
            preferred_element_type=jnp.float32)
        pb = pb + scol * b2_ref[0, :, m * DS:(m + 1) * DS]
        sl = slice(m * DS, (m + 1) * DS)

        @pl.when(first)
        def _init(pb=pb, sl=sl):
            out_ref[:, sl] = pb

        @pl.when(jnp.logical_not(first))
        def _acc(pb=pb, sl=sl):
            out_ref[:, sl] = out_ref[:, sl] + pb


@jax.jit
def _moe(x, Wg, bg2, W1, b1, W2, b2):
    w1_specs = [
        pl.BlockSpec((1, HS1, D), lambda e, k=k: (e, k, 0)) for k in range(S1)
    ]
    w2_specs = [
        pl.BlockSpec((1, DS, H), lambda e, m=m: (e, m, 0)) for m in range(S2)
    ]
    return pl.pallas_call(
        _moe_kernel,
        grid=(E,),
        in_specs=[
            pl.BlockSpec((N, D), lambda e: (0, 0)),
            pl.BlockSpec((E, D), lambda e: (0, 0)),
            pl.BlockSpec((1, E), lambda e: (0, 0)),
            pl.BlockSpec((1, 1, H), lambda e: (e, 0, 0)),
            pl.BlockSpec((1, 1, D), lambda e: (e, 0, 0)),
            *w1_specs,
            *w2_specs,
        ],
        out_specs=pl.BlockSpec((N, D), lambda e: (0, 0)),
        out_shape=jax.ShapeDtypeStruct((N, D), jnp.float32),
        scratch_shapes=[
            pltpu.VMEM((N, H), jnp.bfloat16),
            pltpu.VMEM((N, 1), jnp.float32),
            pltpu.VMEM((N, 1), jnp.float32),
            pltpu.VMEM((N, 1), jnp.int32),
            pltpu.VMEM((N, 1), jnp.int32),
        ],
        compiler_params=pltpu.CompilerParams(
            dimension_semantics=("arbitrary",),
        ),
    )(x, Wg, bg2, b1, b2, *([W1] * S1), *([W2] * S2))


def kernel(x, Wg, bg, W1, b1, W2, b2):
    return _moe(x, Wg, bg.reshape(1, E),
                W1, b1.reshape(E, 1, H), W2, b2.reshape(E, 1, D))


# speedup vs baseline: 1.0489x; 1.0406x over previous
import jax
import jax.numpy as jnp
from jax import lax
from jax.experimental import pallas as pl
from jax.experimental.pallas import tpu as pltpu

E = 16
D = 768
H = 3072
N = 32
NJ = 1
SPLIT = 8
HS = H // (NJ * SPLIT)


def _moe_kernel(*refs):
    (x_ref, wg_ref, bg_ref, b1_ref, b2_ref), w_refs, (out_ref,), \
        (w1s_ref, w2s_ref, i1s_ref, i2s_ref) = \
        refs[:5], refs[5:5 + 2 * SPLIT], refs[5 + 2 * SPLIT:6 + 2 * SPLIT], \
        refs[6 + 2 * SPLIT:]
    w1_refs = w_refs[:SPLIT]
    w2_refs = w_refs[SPLIT:]
    e = pl.program_id(0)
    j = pl.program_id(1)
    first = jnp.logical_and(e == 0, j == 0)

    @pl.when(first)
    def _gate():
        logits = lax.dot_general(
            x_ref[...], wg_ref[...], (((1,), (1,)), ((), ())),
            preferred_element_type=jnp.float32) + bg_ref[...]
        col = lax.broadcasted_iota(jnp.int32, (N, E), 1)
        m1 = jnp.max(logits, axis=-1, keepdims=True)
        i1 = jnp.min(jnp.where(logits == m1, col, E), axis=-1, keepdims=True)
        masked = jnp.where(col == i1, -jnp.inf, logits)
        m2 = jnp.max(masked, axis=-1, keepdims=True)
        i2 = jnp.min(jnp.where(masked == m2, col, E), axis=-1, keepdims=True)
        w1 = 1.0 / (1.0 + jnp.exp(m2 - m1))
        w1s_ref[...] = w1
        w2s_ref[...] = 1.0 - w1
        i1s_ref[...] = i1
        i2s_ref[...] = i2

    scol = (w1s_ref[...] * (i1s_ref[...] == e).astype(jnp.float32)
            + w2s_ref[...] * (i2s_ref[...] == e).astype(jnp.float32))

    xb = x_ref[...].astype(jnp.bfloat16)
    acc = jnp.where(j == 0, scol * b2_ref[0], jnp.zeros((N, D), jnp.float32))
    for k in range(SPLIT):
        h = lax.dot_general(
            xb, w1_refs[k][0].astype(jnp.bfloat16), (((1,), (1,)), ((), ())),
            preferred_element_type=jnp.float32) + b1_ref[0, k:k + 1, :]
        h = jnp.maximum(h, 0.0)
        hs = (h * scol).astype(jnp.bfloat16)
        acc = acc + lax.dot_general(
            hs, w2_refs[k][0].astype(jnp.bfloat16), (((1,), (1,)), ((), ())),
            preferred_element_type=jnp.float32)

    @pl.when(first)
    def _init():
        out_ref[...] = acc

    @pl.when(jnp.logical_not(first))
    def _acc():
        out_ref[...] = out_ref[...] + acc


@jax.jit
def _moe(x, Wg, bg2, W1, b1, W2, b2):
    w1_specs = [
        pl.BlockSpec((1, HS, D), lambda e, j, k=k: (e, j * SPLIT + k, 0))
        for k in range(SPLIT)
    ]
    w2_specs = [
        pl.BlockSpec((1, D, HS), lambda e, j, k=k: (e, 0, j * SPLIT + k))
        for k in range(SPLIT)
    ]
    return pl.pallas_call(
        _moe_kernel,
        grid=(E, NJ),
        in_specs=[
            pl.BlockSpec((N, D), lambda e, j: (0, 0)),
            pl.BlockSpec((E, D), lambda e, j: (0, 0)),
            pl.BlockSpec((1, E), lambda e, j: (0, 0)),
            pl.BlockSpec((1, SPLIT, HS), lambda e, j: (e * NJ + j, 0, 0)),
            pl.BlockSpec((1, 1, D), lambda e, j: (e, 0, 0)),
            *w1_specs,
            *w2_specs,
        ],
        out_specs=pl.BlockSpec((N, D), lambda e, j: (0, 0)),
        out_shape=jax.ShapeDtypeStruct((N, D), jnp.float32),
        scratch_shapes=[
            pltpu.VMEM((N, 1), jnp.float32),
            pltpu.VMEM((N, 1), jnp.float32),
            pltpu.VMEM((N, 1), jnp.int32),
            pltpu.VMEM((N, 1), jnp.int32),
        ],
        compiler_params=pltpu.CompilerParams(
            dimension_semantics=("arbitrary", "arbitrary"),
        ),
    )(x, Wg, bg2, b1, b2, *([W1] * SPLIT), *([W2] * SPLIT))


def kernel(x, Wg, bg, W1, b1, W2, b2):
    return _moe(x, Wg, bg.reshape(1, E),
                W1, b1.reshape(E * NJ, SPLIT, HS),
                W2, b2.reshape(E, 1, D))
